# baseline (speedup 1.0000x reference)
import functools

import jax
import jax.numpy as jnp
from jax import lax
from jax.experimental import pallas as pl
from jax.experimental.pallas import tpu as pltpu
from jax.experimental.pallas import tpu_sc as plsc

K = 16
DIM = 32
BATCH = 4096
NUM_REL = 64
H = 16
NC, NS = 2, 16
NW = NC * NS
BPW = BATCH // NW
HC = 512
NHC = BPW * K // HC


def _sigmoid(v):
    return 1.0 / (1.0 + jnp.exp(-v))


def _recip2(a, b):
    inv = 1.0 / (a * b)
    return inv * b, inv * a


def _recip4(a, b, c, d):
    ab = a * b
    cd = c * d
    inv = 1.0 / (ab * cd)
    iab = inv * cd
    icd = inv * ab
    return iab * b, iab * a, icd * d, icd * c


def _tanh(v):
    return 2.0 / (1.0 + jnp.exp(-2.0 * v)) - 1.0


def _mkgcn_body(users_h, items_h, ent_h, rtt_h, adje_h, adjr_h, hist_h, w_h,
                bias_h, out_h,
                users_v, items_v, hist_v, histf_v, he_v, uemb_v, rtt_v, w_v,
                bias_v, s_v, e1_v, r0_v, v0_v, v1b_v, e2b_v, r1b_v, e2f_v,
                v2b_v, xbuf_v, v1p_v, v0f_v, outbuf_v,
                sem_he, sem_v1, sem_e2, sem_r1, sem_v2, sem0):
    wid = lax.axis_index("s") * NC + lax.axis_index("c")
    base = wid * BPW
    iota16 = lax.iota(jnp.int32, 16)

    pltpu.sync_copy(w_h, w_v)
    pltpu.sync_copy(rtt_h, rtt_v)
    pltpu.sync_copy(bias_h, bias_v)
    pltpu.sync_copy(users_h.at[pl.ds(base, BPW)], users_v)
    pltpu.sync_copy(items_h.at[pl.ds(base, BPW)], items_v)

    pltpu.async_copy(hist_h.at[users_v], hist_v, sem0).wait()

    def flat_body(j, _):
        histf_v[pl.ds(j * K, K)] = hist_v[j, 0:K]
        return 0

    lax.fori_loop(0, BPW, flat_body, 0)

    pltpu.async_copy(ent_h.at[histf_v.at[pl.ds(0, HC)]], he_v.at[0],
                     sem_he.at[0])
    for c in range(NHC):
        cs = c % 2
        if c + 1 < NHC:
            pltpu.async_copy(ent_h.at[histf_v.at[pl.ds((c + 1) * HC, HC)]],
                             he_v.at[1 - cs], sem_he.at[1 - cs])
        pltpu.make_async_copy(ent_h.at[histf_v.at[pl.ds(c * HC, HC)]],
                              he_v.at[cs], sem_he.at[cs]).wait()

        def red_body(j, _, c=c, cs=cs):
            a0 = jnp.zeros((H,), jnp.float32)
            a1 = jnp.zeros((H,), jnp.float32)
            for k in range(K):
                a0 = a0 + he_v[cs, j * K + k, 0:H]
                a1 = a1 + he_v[cs, j * K + k, H:DIM]
            bb = c * (HC // K) + j
            uemb_v[pl.ds(bb * DIM, H)] = a0 * (1.0 / K)
            uemb_v[pl.ds(bb * DIM + H, H)] = a1 * (1.0 / K)
            return 0

        lax.fori_loop(0, HC // K, red_body, 0)

    for rg in range(NUM_REL // H):
        rtt_regs = [rtt_v[d, rg * H:(rg + 1) * H] for d in range(DIM)]

        def s_body(b, _, rg=rg, rtt_regs=rtt_regs):
            u0 = uemb_v[pl.ds(b * DIM, H)]
            u1 = uemb_v[pl.ds(b * DIM + H, H)]
            acc = jnp.zeros((H,), jnp.float32)
            for d in range(H):
                acc = acc + u0[d] * rtt_regs[d]
            for d in range(H):
                acc = acc + u1[d] * rtt_regs[H + d]
            s_v[pl.ds(b * NUM_REL + rg * H, H)] = acc
            return 0

        lax.fori_loop(0, BPW, s_body, 0)

    pltpu.async_copy(adje_h.at[items_v], e1_v, sem0).wait()
    pltpu.async_copy(adjr_h.at[items_v], r0_v, sem0).wait()
    pltpu.async_copy(ent_h.at[items_v], v0_v, sem0).wait()

    bias0 = bias_v[0:H]
    bias1 = bias_v[H:DIM]

    def matmul32(x0, x1):
        p0 = [jnp.zeros((H,), jnp.float32) for _ in range(4)]
        p1 = [jnp.zeros((H,), jnp.float32) for _ in range(4)]
        for kin in range(H):
            xk = x0[kin]
            p0[kin % 4] = p0[kin % 4] + xk * w_v[kin, 0:H]
            p1[kin % 4] = p1[kin % 4] + xk * w_v[kin, H:DIM]
        for kin in range(H):
            xk = x1[kin]
            p0[kin % 4] = p0[kin % 4] + xk * w_v[H + kin, 0:H]
            p1[kin % 4] = p1[kin % 4] + xk * w_v[H + kin, H:DIM]
        return ((p0[0] + p0[1]) + (p0[2] + p0[3]),
                (p1[0] + p1[1]) + (p1[2] + p1[3]))

    def issue_stage_a(b, slot):
        e1row = e1_v.at[b]
        pltpu.async_copy(ent_h.at[e1row], v1b_v.at[slot], sem_v1.at[slot])
        pltpu.async_copy(adje_h.at[e1row], e2b_v.at[slot], sem_e2.at[slot])
        pltpu.async_copy(adjr_h.at[e1row], r1b_v.at[slot], sem_r1.at[slot])

    def wait_stage_a_v1r1(slot):
        pltpu.make_async_copy(ent_h.at[e1_v.at[0]], v1b_v.at[slot],
                              sem_v1.at[slot]).wait()
        pltpu.make_async_copy(adjr_h.at[e1_v.at[0]], r1b_v.at[slot],
                              sem_r1.at[slot]).wait()

    def wait_e2(slot):
        pltpu.make_async_copy(adje_h.at[e1_v.at[0]], e2b_v.at[slot],
                              sem_e2.at[slot]).wait()

    def flatten_e2_issue_v2(slot):
        def fb(j, _):
            e2f_v[slot, pl.ds(j * K, K)] = e2b_v[slot, j, 0:K]
            return 0

        lax.fori_loop(0, K, fb, 0)
        pltpu.async_copy(ent_h.at[e2f_v.at[slot]], v2b_v.at[slot],
                         sem_v2.at[slot])

    def wait_v2(slot):
        pltpu.make_async_copy(ent_h.at[e2f_v.at[0]], v2b_v.at[slot],
                              sem_v2.at[slot]).wait()

    issue_stage_a(0, 0)
    issue_stage_a(1, 1)
    issue_stage_a(2, 2)
    wait_e2(0)
    flatten_e2_issue_v2(0)
    wait_e2(1)
    flatten_e2_issue_v2(1)

    def b_body(b, _):
        s = b & 3
        t = (b + 2) & 3
        wait_stage_a_v1r1(s)
        wait_v2(s)

        @pl.when(b + 2 < BPW)
        def _():
            wait_e2(t)
            flatten_e2_issue_v2(t)

        @plsc.parallel_loop(0, K // 2, step=1, unroll=2)
        def n_body(m):
            n0 = 2 * m
            n1 = 2 * m + 1
            e_a = jnp.exp(plsc.load_gather(s_v,
                                           [b * NUM_REL + r1b_v[s, n0, 0:16]]))
            e_b = jnp.exp(plsc.load_gather(s_v,
                                           [b * NUM_REL + r1b_v[s, n1, 0:16]]))
            sum_a = jnp.zeros((H,), jnp.float32) + jnp.sum(e_a)
            sum_b = jnp.zeros((H,), jnp.float32) + jnp.sum(e_b)
            inv_a, inv_b = _recip2(sum_a, sum_b)
            p_a = e_a * inv_a
            p_b = e_b * inv_b
            for n, p in ((n0, p_a), (n1, p_b)):
                a0 = jnp.zeros((H,), jnp.float32)
                a1 = jnp.zeros((H,), jnp.float32)
                b0 = jnp.zeros((H,), jnp.float32)
                b1 = jnp.zeros((H,), jnp.float32)
                for k in range(0, K, 2):
                    pk = p[k]
                    qk = p[k + 1]
                    a0 = a0 + pk * v2b_v[s, n * K + k, 0:H]
                    a1 = a1 + pk * v2b_v[s, n * K + k, H:DIM]
                    b0 = b0 + qk * v2b_v[s, n * K + k + 1, 0:H]
                    b1 = b1 + qk * v2b_v[s, n * K + k + 1, H:DIM]
                xbuf_v[pl.ds(n * DIM, H)] = (v1b_v[s, n, 0:H] + a0) + b0
                xbuf_v[pl.ds(n * DIM + H, H)] = (v1b_v[s, n, H:DIM] + a1) + b1

        accs = [jnp.zeros((H,), jnp.float32) for _ in range(2 * K)]
        for kin in range(DIM):
            w0 = w_v[kin, 0:H]
            w1 = w_v[kin, H:DIM]
            xcol = plsc.load_gather(xbuf_v, [iota16 * DIM + kin])
            for row in range(K):
                xk = xcol[row]
                accs[2 * row] = accs[2 * row] + xk * w0
                accs[2 * row + 1] = accs[2 * row + 1] + xk * w1
        dens = []
        for row in range(K):
            dens.append(1.0 + jnp.exp(-(accs[2 * row] + bias0)))
            dens.append(1.0 + jnp.exp(-(accs[2 * row + 1] + bias1)))
        for g in range(K // 2):
            r0_, r1_, r2_, r3_ = _recip4(dens[4 * g], dens[4 * g + 1],
                                         dens[4 * g + 2], dens[4 * g + 3])
            v1p_v[2 * g, 0:H] = r0_
            v1p_v[2 * g, H:DIM] = r1_
            v1p_v[2 * g + 1, 0:H] = r2_
            v1p_v[2 * g + 1, H:DIM] = r3_

        r0row = r0_v[b, 0:16]
        sc0 = plsc.load_gather(s_v, [b * NUM_REL + r0row])
        e0 = jnp.exp(sc0)
        p0 = e0 / jnp.sum(e0)

        a0 = jnp.zeros((H,), jnp.float32)
        a1 = jnp.zeros((H,), jnp.float32)
        c0 = jnp.zeros((H,), jnp.float32)
        c1 = jnp.zeros((H,), jnp.float32)
        for k in range(0, K, 2):
            pk = p0[k]
            qk = p0[k + 1]
            a0 = a0 + pk * v1b_v[s, k, 0:H]
            a1 = a1 + pk * v1b_v[s, k, H:DIM]
            c0 = c0 + qk * v1b_v[s, k + 1, 0:H]
            c1 = c1 + qk * v1b_v[s, k + 1, H:DIM]
        a0 = a0 + c0
        a1 = a1 + c1
        y0, y1 = matmul32(v0_v[b, 0:H] + a0, v0_v[b, H:DIM] + a1)
        d0 = 1.0 + jnp.exp(-(y0 + bias0))
        d1 = 1.0 + jnp.exp(-(y1 + bias1))
        v0p0, v0p1 = _recip2(d0, d1)

        a0 = jnp.zeros((H,), jnp.float32)
        a1 = jnp.zeros((H,), jnp.float32)
        c0 = jnp.zeros((H,), jnp.float32)
        c1 = jnp.zeros((H,), jnp.float32)
        for k in range(0, K, 2):
            pk = p0[k]
            qk = p0[k + 1]
            a0 = a0 + pk * v1p_v[k, 0:H]
            a1 = a1 + pk * v1p_v[k, H:DIM]
            c0 = c0 + qk * v1p_v[k + 1, 0:H]
            c1 = c1 + qk * v1p_v[k + 1, H:DIM]
        a0 = a0 + c0
        a1 = a1 + c1
        y0, y1 = matmul32(v0p0 + a0, v0p1 + a1)
        t0 = 1.0 + jnp.exp(-2.0 * (y0 + bias0))
        t1 = 1.0 + jnp.exp(-2.0 * (y1 + bias1))
        it0, it1 = _recip2(t0, t1)
        v0f_v[pl.ds(b * DIM, H)] = 2.0 * it0 - 1.0
        v0f_v[pl.ds(b * DIM + H, H)] = 2.0 * it1 - 1.0

        @pl.when(b + 3 < BPW)
        def _():
            issue_stage_a(b + 3, (b + 3) & 3)

        return 0

    lax.fori_loop(0, BPW, b_body, 0)

    def g_body(g, _):
        rows = (g * 16 + iota16) * DIM
        acc = jnp.zeros((16,), jnp.float32)
        for d in range(DIM):
            acc = acc + (plsc.load_gather(uemb_v, [rows + d]) *
                         plsc.load_gather(v0f_v, [rows + d]))
        outbuf_v[pl.ds(g * 16, 16)] = _sigmoid(acc)
        return 0

    lax.fori_loop(0, BPW // 16, g_body, 0)
    pltpu.sync_copy(outbuf_v, out_h.at[pl.ds(base, BPW)])


_mkgcn = functools.partial(
    pl.kernel,
    out_type=jax.ShapeDtypeStruct((BATCH,), jnp.float32),
    mesh=plsc.VectorSubcoreMesh(core_axis_name="c", subcore_axis_name="s"),
    compiler_params=pltpu.CompilerParams(needs_layout_passes=False,
                                         use_tc_tiling_on_sc=False),
    scratch_types=[
        pltpu.VMEM((BPW,), jnp.int32),
        pltpu.VMEM((BPW,), jnp.int32),
        pltpu.VMEM((BPW, K), jnp.int32),
        pltpu.VMEM((BPW * K,), jnp.int32),
        pltpu.VMEM((2, HC, DIM), jnp.float32),
        pltpu.VMEM((BPW * DIM,), jnp.float32),
        pltpu.VMEM((DIM, NUM_REL), jnp.float32),
        pltpu.VMEM((DIM, DIM), jnp.float32),
        pltpu.VMEM((DIM,), jnp.float32),
        pltpu.VMEM((BPW * NUM_REL,), jnp.float32),
        pltpu.VMEM((BPW, K), jnp.int32),
        pltpu.VMEM((BPW, K), jnp.int32),
        pltpu.VMEM((BPW, DIM), jnp.float32),
        pltpu.VMEM((4, K, DIM), jnp.float32),
        pltpu.VMEM((4, K, K), jnp.int32),
        pltpu.VMEM((4, K, K), jnp.int32),
        pltpu.VMEM((4, K * K), jnp.int32),
        pltpu.VMEM((4, K * K, DIM), jnp.float32),
        pltpu.VMEM((K * DIM,), jnp.float32),
        pltpu.VMEM((K, DIM), jnp.float32),
        pltpu.VMEM((BPW * DIM,), jnp.float32),
        pltpu.VMEM((BPW,), jnp.float32),
        pltpu.SemaphoreType.DMA((2,)),
        pltpu.SemaphoreType.DMA((4,)),
        pltpu.SemaphoreType.DMA((4,)),
        pltpu.SemaphoreType.DMA((4,)),
        pltpu.SemaphoreType.DMA((4,)),
        pltpu.SemaphoreType.DMA,
    ],
)(_mkgcn_body)


@jax.jit
def kernel(users, items, entity_table, relation_table, adj_entity,
           adj_relation, user_history, W, b):
    rtt = relation_table.T
    return _mkgcn(users, items, entity_table, rtt, adj_entity, adj_relation,
                  user_history, W, b)


# speedup vs baseline: 1.0695x; 1.0695x over previous
import functools

import jax
import jax.numpy as jnp
from jax import lax
from jax.experimental import pallas as pl
from jax.experimental.pallas import tpu as pltpu
from jax.experimental.pallas import tpu_sc as plsc

K = 16
DIM = 32
BATCH = 4096
NUM_REL = 64
H = 16
NC, NS = 2, 16
NW = NC * NS
BPW = BATCH // NW
HC = 512
NHC = BPW * K // HC


def _sigmoid(v):
    return 1.0 / (1.0 + jnp.exp(-v))


def _recip2(a, b):
    inv = 1.0 / (a * b)
    return inv * b, inv * a


def _recip4(a, b, c, d):
    ab = a * b
    cd = c * d
    inv = 1.0 / (ab * cd)
    iab = inv * cd
    icd = inv * ab
    return iab * b, iab * a, icd * d, icd * c


def _tanh(v):
    return 2.0 / (1.0 + jnp.exp(-2.0 * v)) - 1.0


def _mkgcn_body(users_h, items_h, ent_h, rtt_h, adje_h, adjr_h, hist_h, w_h,
                bias_h, out_h,
                users_v, items_v, hist_v, histf_v, he_v, uemb_v, rtt_v, w_v,
                bias_v, s_v, e1_v, r0_v, v0_v, v1b_v, e2b_v, r1b_v, e2f_v,
                v2b_v, xbuf_v, v1p_v, v0f_v, outbuf_v,
                sem_he, sem_v1, sem_e2, sem_r1, sem_v2, sem0):
    wid = lax.axis_index("s") * NC + lax.axis_index("c")
    base = wid * BPW
    iota16 = lax.iota(jnp.int32, 16)

    pltpu.sync_copy(w_h, w_v)
    pltpu.sync_copy(rtt_h, rtt_v)
    pltpu.sync_copy(bias_h, bias_v)
    pltpu.sync_copy(users_h.at[pl.ds(base, BPW)], users_v)
    pltpu.sync_copy(items_h.at[pl.ds(base, BPW)], items_v)

    pltpu.async_copy(hist_h.at[users_v], hist_v, sem0).wait()

    def flat_body(j, _):
        histf_v[pl.ds(j * K, K)] = hist_v[j, 0:K]
        return 0

    lax.fori_loop(0, BPW, flat_body, 0)

    pltpu.async_copy(ent_h.at[histf_v.at[pl.ds(0, HC)]], he_v.at[0],
                     sem_he.at[0])
    for c in range(NHC):
        cs = c % 2
        if c + 1 < NHC:
            pltpu.async_copy(ent_h.at[histf_v.at[pl.ds((c + 1) * HC, HC)]],
                             he_v.at[1 - cs], sem_he.at[1 - cs])
        pltpu.make_async_copy(ent_h.at[histf_v.at[pl.ds(c * HC, HC)]],
                              he_v.at[cs], sem_he.at[cs]).wait()

        def red_body(j, _, c=c, cs=cs):
            a0 = jnp.zeros((H,), jnp.float32)
            a1 = jnp.zeros((H,), jnp.float32)
            for k in range(K):
                a0 = a0 + he_v[cs, j * K + k, 0:H]
                a1 = a1 + he_v[cs, j * K + k, H:DIM]
            bb = c * (HC // K) + j
            uemb_v[pl.ds(bb * DIM, H)] = a0 * (1.0 / K)
            uemb_v[pl.ds(bb * DIM + H, H)] = a1 * (1.0 / K)
            return 0

        lax.fori_loop(0, HC // K, red_body, 0)

    for rg in range(NUM_REL // H):
        rtt_regs = [rtt_v[d, rg * H:(rg + 1) * H] for d in range(DIM)]

        def s_body(b, _, rg=rg, rtt_regs=rtt_regs):
            u0 = uemb_v[pl.ds(b * DIM, H)]
            u1 = uemb_v[pl.ds(b * DIM + H, H)]
            acc = jnp.zeros((H,), jnp.float32)
            for d in range(H):
                acc = acc + u0[d] * rtt_regs[d]
            for d in range(H):
                acc = acc + u1[d] * rtt_regs[H + d]
            s_v[pl.ds(b * NUM_REL + rg * H, H)] = acc
            return 0

        lax.fori_loop(0, BPW, s_body, 0)

    pltpu.async_copy(adje_h.at[items_v], e1_v, sem0).wait()
    pltpu.async_copy(adjr_h.at[items_v], r0_v, sem0).wait()
    pltpu.async_copy(ent_h.at[items_v], v0_v, sem0).wait()

    bias0 = bias_v[0:H]
    bias1 = bias_v[H:DIM]

    def matmul32(x0, x1):
        p0 = [jnp.zeros((H,), jnp.float32) for _ in range(4)]
        p1 = [jnp.zeros((H,), jnp.float32) for _ in range(4)]
        for kin in range(H):
            xk = x0[kin]
            p0[kin % 4] = p0[kin % 4] + xk * w_v[kin, 0:H]
            p1[kin % 4] = p1[kin % 4] + xk * w_v[kin, H:DIM]
        for kin in range(H):
            xk = x1[kin]
            p0[kin % 4] = p0[kin % 4] + xk * w_v[H + kin, 0:H]
            p1[kin % 4] = p1[kin % 4] + xk * w_v[H + kin, H:DIM]
        return ((p0[0] + p0[1]) + (p0[2] + p0[3]),
                (p1[0] + p1[1]) + (p1[2] + p1[3]))

    def issue_stage_a(b, slot):
        e1row = e1_v.at[b]
        pltpu.async_copy(ent_h.at[e1row], v1b_v.at[slot], sem_v1.at[slot])
        pltpu.async_copy(adje_h.at[e1row], e2b_v.at[slot], sem_e2.at[slot])
        pltpu.async_copy(adjr_h.at[e1row], r1b_v.at[slot], sem_r1.at[slot])

    def wait_stage_a_v1r1(slot):
        pltpu.make_async_copy(ent_h.at[e1_v.at[0]], v1b_v.at[slot],
                              sem_v1.at[slot]).wait()
        pltpu.make_async_copy(adjr_h.at[e1_v.at[0]], r1b_v.at[slot],
                              sem_r1.at[slot]).wait()

    def wait_e2(slot):
        pltpu.make_async_copy(adje_h.at[e1_v.at[0]], e2b_v.at[slot],
                              sem_e2.at[slot]).wait()

    def flatten_e2_issue_v2(slot):
        def fb(j, _):
            e2f_v[slot, pl.ds(j * K, K)] = e2b_v[slot, j, 0:K]
            return 0

        lax.fori_loop(0, K, fb, 0)
        pltpu.async_copy(ent_h.at[e2f_v.at[slot]], v2b_v.at[slot],
                         sem_v2.at[slot])

    def wait_v2(slot):
        pltpu.make_async_copy(ent_h.at[e2f_v.at[0]], v2b_v.at[slot],
                              sem_v2.at[slot]).wait()

    issue_stage_a(0, 0)
    issue_stage_a(1, 1)
    issue_stage_a(2, 2)
    wait_e2(0)
    flatten_e2_issue_v2(0)
    wait_e2(1)
    flatten_e2_issue_v2(1)

    def b_body(b, _):
        s = b & 3
        t = (b + 2) & 3
        wait_stage_a_v1r1(s)
        wait_v2(s)

        @pl.when(b + 2 < BPW)
        def _():
            wait_e2(t)
            flatten_e2_issue_v2(t)

        def n_body(m, _):
            ns = [4 * m, 4 * m + 1, 4 * m + 2, 4 * m + 3]
            es = [jnp.exp(plsc.load_gather(
                s_v, [b * NUM_REL + r1b_v[s, n, 0:16]])) for n in ns]
            sums = [jnp.zeros((H,), jnp.float32) + jnp.sum(e) for e in es]
            invs = _recip4(*sums)
            pvs = [e * inv for e, inv in zip(es, invs)]
            for n, p in zip(ns, pvs):
                a0 = jnp.zeros((H,), jnp.float32)
                a1 = jnp.zeros((H,), jnp.float32)
                b0 = jnp.zeros((H,), jnp.float32)
                b1 = jnp.zeros((H,), jnp.float32)
                for k in range(0, K, 2):
                    pk = p[k]
                    qk = p[k + 1]
                    a0 = a0 + pk * v2b_v[s, n * K + k, 0:H]
                    a1 = a1 + pk * v2b_v[s, n * K + k, H:DIM]
                    b0 = b0 + qk * v2b_v[s, n * K + k + 1, 0:H]
                    b1 = b1 + qk * v2b_v[s, n * K + k + 1, H:DIM]
                xbuf_v[pl.ds(n * DIM, H)] = (v1b_v[s, n, 0:H] + a0) + b0
                xbuf_v[pl.ds(n * DIM + H, H)] = (v1b_v[s, n, H:DIM] + a1) + b1
            return 0

        lax.fori_loop(0, K // 4, n_body, 0)

        accs = [jnp.zeros((H,), jnp.float32) for _ in range(2 * K)]
        for kin in range(DIM):
            w0 = w_v[kin, 0:H]
            w1 = w_v[kin, H:DIM]
            xcol = plsc.load_gather(xbuf_v, [iota16 * DIM + kin])
            for row in range(K):
                xk = xcol[row]
                accs[2 * row] = accs[2 * row] + xk * w0
                accs[2 * row + 1] = accs[2 * row + 1] + xk * w1
        dens = []
        for row in range(K):
            dens.append(1.0 + jnp.exp(-(accs[2 * row] + bias0)))
            dens.append(1.0 + jnp.exp(-(accs[2 * row + 1] + bias1)))
        for g in range(K // 2):
            r0_, r1_, r2_, r3_ = _recip4(dens[4 * g], dens[4 * g + 1],
                                         dens[4 * g + 2], dens[4 * g + 3])
            v1p_v[2 * g, 0:H] = r0_
            v1p_v[2 * g, H:DIM] = r1_
            v1p_v[2 * g + 1, 0:H] = r2_
            v1p_v[2 * g + 1, H:DIM] = r3_

        r0row = r0_v[b, 0:16]
        sc0 = plsc.load_gather(s_v, [b * NUM_REL + r0row])
        e0 = jnp.exp(sc0)
        p0 = e0 / jnp.sum(e0)

        a0 = jnp.zeros((H,), jnp.float32)
        a1 = jnp.zeros((H,), jnp.float32)
        c0 = jnp.zeros((H,), jnp.float32)
        c1 = jnp.zeros((H,), jnp.float32)
        for k in range(0, K, 2):
            pk = p0[k]
            qk = p0[k + 1]
            a0 = a0 + pk * v1b_v[s, k, 0:H]
            a1 = a1 + pk * v1b_v[s, k, H:DIM]
            c0 = c0 + qk * v1b_v[s, k + 1, 0:H]
            c1 = c1 + qk * v1b_v[s, k + 1, H:DIM]
        a0 = a0 + c0
        a1 = a1 + c1
        y0, y1 = matmul32(v0_v[b, 0:H] + a0, v0_v[b, H:DIM] + a1)
        d0 = 1.0 + jnp.exp(-(y0 + bias0))
        d1 = 1.0 + jnp.exp(-(y1 + bias1))
        v0p0, v0p1 = _recip2(d0, d1)

        a0 = jnp.zeros((H,), jnp.float32)
        a1 = jnp.zeros((H,), jnp.float32)
        c0 = jnp.zeros((H,), jnp.float32)
        c1 = jnp.zeros((H,), jnp.float32)
        for k in range(0, K, 2):
            pk = p0[k]
            qk = p0[k + 1]
            a0 = a0 + pk * v1p_v[k, 0:H]
            a1 = a1 + pk * v1p_v[k, H:DIM]
            c0 = c0 + qk * v1p_v[k + 1, 0:H]
            c1 = c1 + qk * v1p_v[k + 1, H:DIM]
        a0 = a0 + c0
        a1 = a1 + c1
        y0, y1 = matmul32(v0p0 + a0, v0p1 + a1)
        t0 = 1.0 + jnp.exp(-2.0 * (y0 + bias0))
        t1 = 1.0 + jnp.exp(-2.0 * (y1 + bias1))
        it0, it1 = _recip2(t0, t1)
        v0f_v[pl.ds(b * DIM, H)] = 2.0 * it0 - 1.0
        v0f_v[pl.ds(b * DIM + H, H)] = 2.0 * it1 - 1.0

        @pl.when(b + 3 < BPW)
        def _():
            issue_stage_a(b + 3, (b + 3) & 3)

        return 0

    lax.fori_loop(0, BPW, b_body, 0)

    def g_body(g, _):
        rows = (g * 16 + iota16) * DIM
        acc = jnp.zeros((16,), jnp.float32)
        for d in range(DIM):
            acc = acc + (plsc.load_gather(uemb_v, [rows + d]) *
                         plsc.load_gather(v0f_v, [rows + d]))
        outbuf_v[pl.ds(g * 16, 16)] = _sigmoid(acc)
        return 0

    lax.fori_loop(0, BPW // 16, g_body, 0)
    pltpu.sync_copy(outbuf_v, out_h.at[pl.ds(base, BPW)])


_mkgcn = functools.partial(
    pl.kernel,
    out_type=jax.ShapeDtypeStruct((BATCH,), jnp.float32),
    mesh=plsc.VectorSubcoreMesh(core_axis_name="c", subcore_axis_name="s"),
    compiler_params=pltpu.CompilerParams(needs_layout_passes=False,
                                         use_tc_tiling_on_sc=False),
    scratch_types=[
        pltpu.VMEM((BPW,), jnp.int32),
        pltpu.VMEM((BPW,), jnp.int32),
        pltpu.VMEM((BPW, K), jnp.int32),
        pltpu.VMEM((BPW * K,), jnp.int32),
        pltpu.VMEM((2, HC, DIM), jnp.float32),
        pltpu.VMEM((BPW * DIM,), jnp.float32),
        pltpu.VMEM((DIM, NUM_REL), jnp.float32),
        pltpu.VMEM((DIM, DIM), jnp.float32),
        pltpu.VMEM((DIM,), jnp.float32),
        pltpu.VMEM((BPW * NUM_REL,), jnp.float32),
        pltpu.VMEM((BPW, K), jnp.int32),
        pltpu.VMEM((BPW, K), jnp.int32),
        pltpu.VMEM((BPW, DIM), jnp.float32),
        pltpu.VMEM((4, K, DIM), jnp.float32),
        pltpu.VMEM((4, K, K), jnp.int32),
        pltpu.VMEM((4, K, K), jnp.int32),
        pltpu.VMEM((4, K * K), jnp.int32),
        pltpu.VMEM((4, K * K, DIM), jnp.float32),
        pltpu.VMEM((K * DIM,), jnp.float32),
        pltpu.VMEM((K, DIM), jnp.float32),
        pltpu.VMEM((BPW * DIM,), jnp.float32),
        pltpu.VMEM((BPW,), jnp.float32),
        pltpu.SemaphoreType.DMA((2,)),
        pltpu.SemaphoreType.DMA((4,)),
        pltpu.SemaphoreType.DMA((4,)),
        pltpu.SemaphoreType.DMA((4,)),
        pltpu.SemaphoreType.DMA((4,)),
        pltpu.SemaphoreType.DMA,
    ],
)(_mkgcn_body)


@jax.jit
def kernel(users, items, entity_table, relation_table, adj_entity,
           adj_relation, user_history, W, b):
    rtt = relation_table.T
    return _mkgcn(users, items, entity_table, rtt, adj_entity, adj_relation,
                  user_history, W, b)


# speedup vs baseline: 1.2982x; 1.2139x over previous
import functools

import jax
import jax.numpy as jnp
from jax import lax
from jax.experimental import pallas as pl
from jax.experimental.pallas import tpu as pltpu
from jax.experimental.pallas import tpu_sc as plsc

K = 16
DIM = 32
BATCH = 4096
NUM_REL = 64
H = 16
NC, NS = 2, 16
NW = NC * NS
BPW = BATCH // NW
HC = 512
NHC = BPW * K // HC


def _sigmoid(v):
    return 1.0 / (1.0 + jnp.exp(-v))


def _recip2(a, b):
    inv = 1.0 / (a * b)
    return inv * b, inv * a


def _recip4(a, b, c, d):
    ab = a * b
    cd = c * d
    inv = 1.0 / (ab * cd)
    iab = inv * cd
    icd = inv * ab
    return iab * b, iab * a, icd * d, icd * c


def _tanh(v):
    return 2.0 / (1.0 + jnp.exp(-2.0 * v)) - 1.0


def _mkgcn_body(users_h, items_h, ent_h, rtt_h, adje_h, adjr_h, hist_h, w_h,
                bias_h, out_h,
                users_v, items_v, hist_v, histf_v, he_v, uemb_v, rtt_v, w_v,
                bias_v, s_v, e1_v, r0_v, v0_v, v1b_v, e2b_v, r1b_v, e2f_v,
                v2b_v, xbuf_v, v1p_v, v0f_v, outbuf_v,
                sem_he, sem_v1, sem_e2, sem_r1, sem_v2, sem0):
    wid = lax.axis_index("s") * NC + lax.axis_index("c")
    base = wid * BPW
    iota16 = lax.iota(jnp.int32, 16)

    pltpu.sync_copy(w_h, w_v)
    pltpu.sync_copy(rtt_h, rtt_v)
    pltpu.sync_copy(bias_h, bias_v)
    pltpu.sync_copy(users_h.at[pl.ds(base, BPW)], users_v)
    pltpu.sync_copy(items_h.at[pl.ds(base, BPW)], items_v)

    pltpu.async_copy(hist_h.at[users_v], hist_v, sem0).wait()

    def flat_body(j, _):
        histf_v[pl.ds(j * K, K)] = hist_v[j, 0:K]
        return 0

    lax.fori_loop(0, BPW, flat_body, 0)

    pltpu.async_copy(ent_h.at[histf_v.at[pl.ds(0, HC)]], he_v.at[0],
                     sem_he.at[0])
    for c in range(NHC):
        cs = c % 2
        if c + 1 < NHC:
            pltpu.async_copy(ent_h.at[histf_v.at[pl.ds((c + 1) * HC, HC)]],
                             he_v.at[1 - cs], sem_he.at[1 - cs])
        pltpu.make_async_copy(ent_h.at[histf_v.at[pl.ds(c * HC, HC)]],
                              he_v.at[cs], sem_he.at[cs]).wait()

        def red_body(j, _, c=c, cs=cs):
            a0 = jnp.zeros((H,), jnp.float32)
            a1 = jnp.zeros((H,), jnp.float32)
            for k in range(K):
                a0 = a0 + he_v[cs, j * K + k, 0:H]
                a1 = a1 + he_v[cs, j * K + k, H:DIM]
            bb = c * (HC // K) + j
            uemb_v[pl.ds(bb * DIM, H)] = a0 * (1.0 / K)
            uemb_v[pl.ds(bb * DIM + H, H)] = a1 * (1.0 / K)
            return 0

        lax.fori_loop(0, HC // K, red_body, 0)

    for rg in range(NUM_REL // H):
        rtt_regs = [rtt_v[d, rg * H:(rg + 1) * H] for d in range(DIM)]

        def s_body(b, _, rg=rg, rtt_regs=rtt_regs):
            u0 = uemb_v[pl.ds(b * DIM, H)]
            u1 = uemb_v[pl.ds(b * DIM + H, H)]
            acc = jnp.zeros((H,), jnp.float32)
            for d in range(H):
                acc = acc + u0[d] * rtt_regs[d]
            for d in range(H):
                acc = acc + u1[d] * rtt_regs[H + d]
            s_v[pl.ds(b * NUM_REL + rg * H, H)] = acc
            return 0

        lax.fori_loop(0, BPW, s_body, 0)

    pltpu.async_copy(adje_h.at[items_v], e1_v, sem0).wait()
    pltpu.async_copy(adjr_h.at[items_v], r0_v, sem0).wait()
    pltpu.async_copy(ent_h.at[items_v], v0_v, sem0).wait()

    bias0 = bias_v[0:H]
    bias1 = bias_v[H:DIM]

    def matmul32(x0, x1):
        p0 = [jnp.zeros((H,), jnp.float32) for _ in range(4)]
        p1 = [jnp.zeros((H,), jnp.float32) for _ in range(4)]
        for kin in range(H):
            xk = x0[kin]
            p0[kin % 4] = p0[kin % 4] + xk * w_v[kin, 0:H]
            p1[kin % 4] = p1[kin % 4] + xk * w_v[kin, H:DIM]
        for kin in range(H):
            xk = x1[kin]
            p0[kin % 4] = p0[kin % 4] + xk * w_v[H + kin, 0:H]
            p1[kin % 4] = p1[kin % 4] + xk * w_v[H + kin, H:DIM]
        return ((p0[0] + p0[1]) + (p0[2] + p0[3]),
                (p1[0] + p1[1]) + (p1[2] + p1[3]))

    def issue_stage_a(b, slot):
        e1row = e1_v.at[b]
        pltpu.async_copy(ent_h.at[e1row], v1b_v.at[slot], sem_v1.at[slot])
        pltpu.async_copy(adje_h.at[e1row], e2b_v.at[slot], sem_e2.at[slot])
        pltpu.async_copy(adjr_h.at[e1row], r1b_v.at[slot], sem_r1.at[slot])

    def wait_stage_a_v1r1(slot):
        pltpu.make_async_copy(ent_h.at[e1_v.at[0]], v1b_v.at[slot],
                              sem_v1.at[slot]).wait()
        pltpu.make_async_copy(adjr_h.at[e1_v.at[0]], r1b_v.at[slot],
                              sem_r1.at[slot]).wait()

    def wait_e2(slot):
        pltpu.make_async_copy(adje_h.at[e1_v.at[0]], e2b_v.at[slot],
                              sem_e2.at[slot]).wait()

    def flatten_e2_issue_v2(slot):
        def fb(j, _):
            e2f_v[slot, pl.ds(j * K, K)] = e2b_v[slot, j, 0:K]
            return 0

        lax.fori_loop(0, K, fb, 0)
        pltpu.async_copy(ent_h.at[e2f_v.at[slot]], v2b_v.at[slot],
                         sem_v2.at[slot])

    def wait_v2(slot):
        pltpu.make_async_copy(ent_h.at[e2f_v.at[0]], v2b_v.at[slot],
                              sem_v2.at[slot]).wait()

    issue_stage_a(0, 0)
    issue_stage_a(1, 1)
    issue_stage_a(2, 2)
    wait_e2(0)
    flatten_e2_issue_v2(0)
    wait_e2(1)
    flatten_e2_issue_v2(1)

    def b_body(b, _):
        s = b & 3
        t = (b + 2) & 3
        wait_stage_a_v1r1(s)
        wait_v2(s)

        @pl.when(b + 2 < BPW)
        def _():
            wait_e2(t)
            flatten_e2_issue_v2(t)

        def n_body(m, _):
            n0 = 2 * m
            n1 = 2 * m + 1
            e_a = jnp.exp(plsc.load_gather(s_v,
                                           [b * NUM_REL + r1b_v[s, n0, 0:16]]))
            e_b = jnp.exp(plsc.load_gather(s_v,
                                           [b * NUM_REL + r1b_v[s, n1, 0:16]]))
            sum_a = jnp.zeros((H,), jnp.float32) + jnp.sum(e_a)
            sum_b = jnp.zeros((H,), jnp.float32) + jnp.sum(e_b)
            inv_a, inv_b = _recip2(sum_a, sum_b)
            p_a = e_a * inv_a
            p_b = e_b * inv_b
            for n, p in ((n0, p_a), (n1, p_b)):
                a0 = jnp.zeros((H,), jnp.float32)
                a1 = jnp.zeros((H,), jnp.float32)
                b0 = jnp.zeros((H,), jnp.float32)
                b1 = jnp.zeros((H,), jnp.float32)
                for k in range(0, K, 2):
                    pk = p[k]
                    qk = p[k + 1]
                    a0 = a0 + pk * v2b_v[s, n * K + k, 0:H]
                    a1 = a1 + pk * v2b_v[s, n * K + k, H:DIM]
                    b0 = b0 + qk * v2b_v[s, n * K + k + 1, 0:H]
                    b1 = b1 + qk * v2b_v[s, n * K + k + 1, H:DIM]
                xbuf_v[pl.ds(n * DIM, H)] = (v1b_v[s, n, 0:H] + a0) + b0
                xbuf_v[pl.ds(n * DIM + H, H)] = (v1b_v[s, n, H:DIM] + a1) + b1
            return 0

        lax.fori_loop(0, K // 2, n_body, 0)

        accs = [jnp.zeros((H,), jnp.float32) for _ in range(2 * K)]
        for kin in range(DIM):
            w0 = w_v[kin, 0:H]
            w1 = w_v[kin, H:DIM]
            xcol = plsc.load_gather(xbuf_v, [iota16 * DIM + kin])
            for row in range(K):
                xk = xcol[row]
                accs[2 * row] = accs[2 * row] + xk * w0
                accs[2 * row + 1] = accs[2 * row + 1] + xk * w1
        dens = []
        for row in range(K):
            dens.append(1.0 + jnp.exp(-(accs[2 * row] + bias0)))
            dens.append(1.0 + jnp.exp(-(accs[2 * row + 1] + bias1)))
        for g in range(K // 2):
            r0_, r1_, r2_, r3_ = _recip4(dens[4 * g], dens[4 * g + 1],
                                         dens[4 * g + 2], dens[4 * g + 3])
            v1p_v[2 * g, 0:H] = r0_
            v1p_v[2 * g, H:DIM] = r1_
            v1p_v[2 * g + 1, 0:H] = r2_
            v1p_v[2 * g + 1, H:DIM] = r3_

        r0row = r0_v[b, 0:16]
        sc0 = plsc.load_gather(s_v, [b * NUM_REL + r0row])
        e0 = jnp.exp(sc0)
        p0 = e0 / jnp.sum(e0)

        a0 = jnp.zeros((H,), jnp.float32)
        a1 = jnp.zeros((H,), jnp.float32)
        c0 = jnp.zeros((H,), jnp.float32)
        c1 = jnp.zeros((H,), jnp.float32)
        for k in range(0, K, 2):
            pk = p0[k]
            qk = p0[k + 1]
            a0 = a0 + pk * v1b_v[s, k, 0:H]
            a1 = a1 + pk * v1b_v[s, k, H:DIM]
            c0 = c0 + qk * v1b_v[s, k + 1, 0:H]
            c1 = c1 + qk * v1b_v[s, k + 1, H:DIM]
        a0 = a0 + c0
        a1 = a1 + c1
        y0, y1 = matmul32(v0_v[b, 0:H] + a0, v0_v[b, H:DIM] + a1)
        d0 = 1.0 + jnp.exp(-(y0 + bias0))
        d1 = 1.0 + jnp.exp(-(y1 + bias1))
        v0p0, v0p1 = _recip2(d0, d1)

        a0 = jnp.zeros((H,), jnp.float32)
        a1 = jnp.zeros((H,), jnp.float32)
        c0 = jnp.zeros((H,), jnp.float32)
        c1 = jnp.zeros((H,), jnp.float32)
        for k in range(0, K, 2):
            pk = p0[k]
            qk = p0[k + 1]
            a0 = a0 + pk * v1p_v[k, 0:H]
            a1 = a1 + pk * v1p_v[k, H:DIM]
            c0 = c0 + qk * v1p_v[k + 1, 0:H]
            c1 = c1 + qk * v1p_v[k + 1, H:DIM]
        a0 = a0 + c0
        a1 = a1 + c1
        y0, y1 = matmul32(v0p0 + a0, v0p1 + a1)
        t0 = 1.0 + jnp.exp(-2.0 * (y0 + bias0))
        t1 = 1.0 + jnp.exp(-2.0 * (y1 + bias1))
        it0, it1 = _recip2(t0, t1)
        v0f_v[pl.ds(b * DIM, H)] = 2.0 * it0 - 1.0
        v0f_v[pl.ds(b * DIM + H, H)] = 2.0 * it1 - 1.0

        @pl.when(b + 3 < BPW)
        def _():
            issue_stage_a(b + 3, (b + 3) & 3)

        return 0

    lax.fori_loop(0, BPW, b_body, 0)

    def g_body(g, _):
        rows = (g * 16 + iota16) * DIM
        acc = jnp.zeros((16,), jnp.float32)
        for d in range(DIM):
            acc = acc + (plsc.load_gather(uemb_v, [rows + d]) *
                         plsc.load_gather(v0f_v, [rows + d]))
        outbuf_v[pl.ds(g * 16, 16)] = _sigmoid(acc)
        return 0

    lax.fori_loop(0, BPW // 16, g_body, 0)
    pltpu.sync_copy(outbuf_v, out_h.at[pl.ds(base, BPW)])


_mkgcn = functools.partial(
    pl.kernel,
    out_type=jax.ShapeDtypeStruct((BATCH,), jnp.float32),
    mesh=plsc.VectorSubcoreMesh(core_axis_name="c", subcore_axis_name="s"),
    compiler_params=pltpu.CompilerParams(needs_layout_passes=False,
                                         use_tc_tiling_on_sc=False),
    scratch_types=[
        pltpu.VMEM((BPW,), jnp.int32),
        pltpu.VMEM((BPW,), jnp.int32),
        pltpu.VMEM((BPW, K), jnp.int32),
        pltpu.VMEM((BPW * K,), jnp.int32),
        pltpu.VMEM((2, HC, DIM), jnp.float32),
        pltpu.VMEM((BPW * DIM,), jnp.float32),
        pltpu.VMEM((DIM, NUM_REL), jnp.float32),
        pltpu.VMEM((DIM, DIM), jnp.float32),
        pltpu.VMEM((DIM,), jnp.float32),
        pltpu.VMEM((BPW * NUM_REL,), jnp.float32),
        pltpu.VMEM((BPW, K), jnp.int32),
        pltpu.VMEM((BPW, K), jnp.int32),
        pltpu.VMEM((BPW, DIM), jnp.float32),
        pltpu.VMEM((4, K, DIM), jnp.float32),
        pltpu.VMEM((4, K, K), jnp.int32),
        pltpu.VMEM((4, K, K), jnp.int32),
        pltpu.VMEM((4, K * K), jnp.int32),
        pltpu.VMEM((4, K * K, DIM), jnp.float32),
        pltpu.VMEM((K * DIM,), jnp.float32),
        pltpu.VMEM((K, DIM), jnp.float32),
        pltpu.VMEM((BPW * DIM,), jnp.float32),
        pltpu.VMEM((BPW,), jnp.float32),
        pltpu.SemaphoreType.DMA((2,)),
        pltpu.SemaphoreType.DMA((4,)),
        pltpu.SemaphoreType.DMA((4,)),
        pltpu.SemaphoreType.DMA((4,)),
        pltpu.SemaphoreType.DMA((4,)),
        pltpu.SemaphoreType.DMA,
    ],
)(_mkgcn_body)


@jax.jit
def kernel(users, items, entity_table, relation_table, adj_entity,
           adj_relation, user_history, W, b):
    rtt = relation_table.T
    return _mkgcn(users, items, entity_table, rtt, adj_entity, adj_relation,
                  user_history, W, b)


# speedup vs baseline: 1.3027x; 1.0034x over previous
import functools

import jax
import jax.numpy as jnp
from jax import lax
from jax.experimental import pallas as pl
from jax.experimental.pallas import tpu as pltpu
from jax.experimental.pallas import tpu_sc as plsc

K = 16
DIM = 32
BATCH = 4096
NUM_REL = 64
H = 16
NC, NS = 2, 16
NW = NC * NS
BPW = BATCH // NW
HC = 512
NHC = BPW * K // HC


def _sigmoid(v):
    return 1.0 / (1.0 + jnp.exp(-v))


def _recip2(a, b):
    inv = 1.0 / (a * b)
    return inv * b, inv * a


def _recip4(a, b, c, d):
    ab = a * b
    cd = c * d
    inv = 1.0 / (ab * cd)
    iab = inv * cd
    icd = inv * ab
    return iab * b, iab * a, icd * d, icd * c


def _tanh(v):
    return 2.0 / (1.0 + jnp.exp(-2.0 * v)) - 1.0


def _mkgcn_body(users_h, items_h, ent_h, rtt_h, adje_h, adjr_h, hist_h, w_h,
                bias_h, out_h,
                users_v, items_v, hist_v, histf_v, he_v, uemb_v, rtt_v, w_v,
                bias_v, s_v, e1_v, r0_v, v0_v, v1b_v, e2b_v, r1b_v, e2f_v,
                v2b_v, xbuf_v, v1p_v, v0f_v, outbuf_v,
                sem_he, sem_v1, sem_e2, sem_r1, sem_v2, sem0):
    wid = lax.axis_index("s") * NC + lax.axis_index("c")
    base = wid * BPW
    iota16 = lax.iota(jnp.int32, 16)

    pltpu.sync_copy(w_h, w_v)
    pltpu.sync_copy(rtt_h, rtt_v)
    pltpu.sync_copy(bias_h, bias_v)
    pltpu.sync_copy(users_h.at[pl.ds(base, BPW)], users_v)
    pltpu.sync_copy(items_h.at[pl.ds(base, BPW)], items_v)

    pltpu.async_copy(hist_h.at[users_v], hist_v, sem0)
    pltpu.async_copy(adje_h.at[items_v], e1_v, sem_v1.at[0])
    pltpu.async_copy(adjr_h.at[items_v], r0_v, sem_r1.at[0])
    pltpu.async_copy(ent_h.at[items_v], v0_v, sem_v2.at[0])

    pltpu.make_async_copy(hist_h.at[users_v], hist_v, sem0).wait()

    def flat_body(j, _):
        histf_v[pl.ds(j * K, K)] = hist_v[j, 0:K]
        return 0

    lax.fori_loop(0, BPW, flat_body, 0)

    pltpu.async_copy(ent_h.at[histf_v.at[pl.ds(0, HC)]], he_v.at[0],
                     sem_he.at[0])
    for c in range(NHC):
        cs = c % 2
        if c + 1 < NHC:
            pltpu.async_copy(ent_h.at[histf_v.at[pl.ds((c + 1) * HC, HC)]],
                             he_v.at[1 - cs], sem_he.at[1 - cs])
        pltpu.make_async_copy(ent_h.at[histf_v.at[pl.ds(c * HC, HC)]],
                              he_v.at[cs], sem_he.at[cs]).wait()

        def red_body(j, _, c=c, cs=cs):
            a0 = jnp.zeros((H,), jnp.float32)
            a1 = jnp.zeros((H,), jnp.float32)
            for k in range(K):
                a0 = a0 + he_v[cs, j * K + k, 0:H]
                a1 = a1 + he_v[cs, j * K + k, H:DIM]
            bb = c * (HC // K) + j
            uemb_v[pl.ds(bb * DIM, H)] = a0 * (1.0 / K)
            uemb_v[pl.ds(bb * DIM + H, H)] = a1 * (1.0 / K)
            return 0

        lax.fori_loop(0, HC // K, red_body, 0)

    for rg in range(NUM_REL // H):
        rtt_regs = [rtt_v[d, rg * H:(rg + 1) * H] for d in range(DIM)]

        def s_body(b, _, rg=rg, rtt_regs=rtt_regs):
            u0 = uemb_v[pl.ds(b * DIM, H)]
            u1 = uemb_v[pl.ds(b * DIM + H, H)]
            acc = jnp.zeros((H,), jnp.float32)
            for d in range(H):
                acc = acc + u0[d] * rtt_regs[d]
            for d in range(H):
                acc = acc + u1[d] * rtt_regs[H + d]
            s_v[pl.ds(b * NUM_REL + rg * H, H)] = acc
            return 0

        lax.fori_loop(0, BPW, s_body, 0)

    pltpu.make_async_copy(adje_h.at[items_v], e1_v, sem_v1.at[0]).wait()
    pltpu.make_async_copy(adjr_h.at[items_v], r0_v, sem_r1.at[0]).wait()
    pltpu.make_async_copy(ent_h.at[items_v], v0_v, sem_v2.at[0]).wait()

    bias0 = bias_v[0:H]
    bias1 = bias_v[H:DIM]

    def matmul32(x0, x1):
        p0 = [jnp.zeros((H,), jnp.float32) for _ in range(4)]
        p1 = [jnp.zeros((H,), jnp.float32) for _ in range(4)]
        for kin in range(H):
            xk = x0[kin]
            p0[kin % 4] = p0[kin % 4] + xk * w_v[kin, 0:H]
            p1[kin % 4] = p1[kin % 4] + xk * w_v[kin, H:DIM]
        for kin in range(H):
            xk = x1[kin]
            p0[kin % 4] = p0[kin % 4] + xk * w_v[H + kin, 0:H]
            p1[kin % 4] = p1[kin % 4] + xk * w_v[H + kin, H:DIM]
        return ((p0[0] + p0[1]) + (p0[2] + p0[3]),
                (p1[0] + p1[1]) + (p1[2] + p1[3]))

    def issue_stage_a(b, slot):
        e1row = e1_v.at[b]
        pltpu.async_copy(ent_h.at[e1row], v1b_v.at[slot], sem_v1.at[slot])
        pltpu.async_copy(adje_h.at[e1row], e2b_v.at[slot], sem_e2.at[slot])
        pltpu.async_copy(adjr_h.at[e1row], r1b_v.at[slot], sem_r1.at[slot])

    def wait_stage_a_v1r1(slot):
        pltpu.make_async_copy(ent_h.at[e1_v.at[0]], v1b_v.at[slot],
                              sem_v1.at[slot]).wait()
        pltpu.make_async_copy(adjr_h.at[e1_v.at[0]], r1b_v.at[slot],
                              sem_r1.at[slot]).wait()

    def wait_e2(slot):
        pltpu.make_async_copy(adje_h.at[e1_v.at[0]], e2b_v.at[slot],
                              sem_e2.at[slot]).wait()

    def flatten_e2_issue_v2(slot):
        def fb(j, _):
            e2f_v[slot, pl.ds(j * K, K)] = e2b_v[slot, j, 0:K]
            return 0

        lax.fori_loop(0, K, fb, 0)
        pltpu.async_copy(ent_h.at[e2f_v.at[slot]], v2b_v.at[slot],
                         sem_v2.at[slot])

    def wait_v2(slot):
        pltpu.make_async_copy(ent_h.at[e2f_v.at[0]], v2b_v.at[slot],
                              sem_v2.at[slot]).wait()

    issue_stage_a(0, 0)
    issue_stage_a(1, 1)
    issue_stage_a(2, 2)
    wait_e2(0)
    flatten_e2_issue_v2(0)
    wait_e2(1)
    flatten_e2_issue_v2(1)

    def b_body(b, _):
        s = b & 3
        t = (b + 2) & 3
        wait_stage_a_v1r1(s)
        wait_v2(s)

        @pl.when(b + 2 < BPW)
        def _():
            wait_e2(t)
            flatten_e2_issue_v2(t)

        def n_body(m, _):
            n0 = 2 * m
            n1 = 2 * m + 1
            e_a = jnp.exp(plsc.load_gather(s_v,
                                           [b * NUM_REL + r1b_v[s, n0, 0:16]]))
            e_b = jnp.exp(plsc.load_gather(s_v,
                                           [b * NUM_REL + r1b_v[s, n1, 0:16]]))
            sum_a = jnp.zeros((H,), jnp.float32) + jnp.sum(e_a)
            sum_b = jnp.zeros((H,), jnp.float32) + jnp.sum(e_b)
            inv_a, inv_b = _recip2(sum_a, sum_b)
            p_a = e_a * inv_a
            p_b = e_b * inv_b
            for n, p in ((n0, p_a), (n1, p_b)):
                a0 = jnp.zeros((H,), jnp.float32)
                a1 = jnp.zeros((H,), jnp.float32)
                b0 = jnp.zeros((H,), jnp.float32)
                b1 = jnp.zeros((H,), jnp.float32)
                for k in range(0, K, 2):
                    pk = p[k]
                    qk = p[k + 1]
                    a0 = a0 + pk * v2b_v[s, n * K + k, 0:H]
                    a1 = a1 + pk * v2b_v[s, n * K + k, H:DIM]
                    b0 = b0 + qk * v2b_v[s, n * K + k + 1, 0:H]
                    b1 = b1 + qk * v2b_v[s, n * K + k + 1, H:DIM]
                xbuf_v[pl.ds(n * DIM, H)] = (v1b_v[s, n, 0:H] + a0) + b0
                xbuf_v[pl.ds(n * DIM + H, H)] = (v1b_v[s, n, H:DIM] + a1) + b1
            return 0

        lax.fori_loop(0, K // 2, n_body, 0)

        accs = [jnp.zeros((H,), jnp.float32) for _ in range(2 * K)]
        for kin in range(DIM):
            w0 = w_v[kin, 0:H]
            w1 = w_v[kin, H:DIM]
            xcol = plsc.load_gather(xbuf_v, [iota16 * DIM + kin])
            for row in range(K):
                xk = xcol[row]
                accs[2 * row] = accs[2 * row] + xk * w0
                accs[2 * row + 1] = accs[2 * row + 1] + xk * w1
        dens = []
        for row in range(K):
            dens.append(1.0 + jnp.exp(-(accs[2 * row] + bias0)))
            dens.append(1.0 + jnp.exp(-(accs[2 * row + 1] + bias1)))
        for g in range(K // 2):
            r0_, r1_, r2_, r3_ = _recip4(dens[4 * g], dens[4 * g + 1],
                                         dens[4 * g + 2], dens[4 * g + 3])
            v1p_v[2 * g, 0:H] = r0_
            v1p_v[2 * g, H:DIM] = r1_
            v1p_v[2 * g + 1, 0:H] = r2_
            v1p_v[2 * g + 1, H:DIM] = r3_

        r0row = r0_v[b, 0:16]
        sc0 = plsc.load_gather(s_v, [b * NUM_REL + r0row])
        e0 = jnp.exp(sc0)
        p0 = e0 / jnp.sum(e0)

        a0 = jnp.zeros((H,), jnp.float32)
        a1 = jnp.zeros((H,), jnp.float32)
        c0 = jnp.zeros((H,), jnp.float32)
        c1 = jnp.zeros((H,), jnp.float32)
        for k in range(0, K, 2):
            pk = p0[k]
            qk = p0[k + 1]
            a0 = a0 + pk * v1b_v[s, k, 0:H]
            a1 = a1 + pk * v1b_v[s, k, H:DIM]
            c0 = c0 + qk * v1b_v[s, k + 1, 0:H]
            c1 = c1 + qk * v1b_v[s, k + 1, H:DIM]
        a0 = a0 + c0
        a1 = a1 + c1
        y0, y1 = matmul32(v0_v[b, 0:H] + a0, v0_v[b, H:DIM] + a1)
        d0 = 1.0 + jnp.exp(-(y0 + bias0))
        d1 = 1.0 + jnp.exp(-(y1 + bias1))
        v0p0, v0p1 = _recip2(d0, d1)

        a0 = jnp.zeros((H,), jnp.float32)
        a1 = jnp.zeros((H,), jnp.float32)
        c0 = jnp.zeros((H,), jnp.float32)
        c1 = jnp.zeros((H,), jnp.float32)
        for k in range(0, K, 2):
            pk = p0[k]
            qk = p0[k + 1]
            a0 = a0 + pk * v1p_v[k, 0:H]
            a1 = a1 + pk * v1p_v[k, H:DIM]
            c0 = c0 + qk * v1p_v[k + 1, 0:H]
            c1 = c1 + qk * v1p_v[k + 1, H:DIM]
        a0 = a0 + c0
        a1 = a1 + c1
        y0, y1 = matmul32(v0p0 + a0, v0p1 + a1)
        t0 = 1.0 + jnp.exp(-2.0 * (y0 + bias0))
        t1 = 1.0 + jnp.exp(-2.0 * (y1 + bias1))
        it0, it1 = _recip2(t0, t1)
        v0f_v[pl.ds(b * DIM, H)] = 2.0 * it0 - 1.0
        v0f_v[pl.ds(b * DIM + H, H)] = 2.0 * it1 - 1.0

        @pl.when(b + 3 < BPW)
        def _():
            issue_stage_a(b + 3, (b + 3) & 3)

        return 0

    lax.fori_loop(0, BPW, b_body, 0)

    def g_body(g, _):
        rows = (g * 16 + iota16) * DIM
        acc = jnp.zeros((16,), jnp.float32)
        for d in range(DIM):
            acc = acc + (plsc.load_gather(uemb_v, [rows + d]) *
                         plsc.load_gather(v0f_v, [rows + d]))
        outbuf_v[pl.ds(g * 16, 16)] = _sigmoid(acc)
        return 0

    lax.fori_loop(0, BPW // 16, g_body, 0)
    pltpu.sync_copy(outbuf_v, out_h.at[pl.ds(base, BPW)])


_mkgcn = functools.partial(
    pl.kernel,
    out_type=jax.ShapeDtypeStruct((BATCH,), jnp.float32),
    mesh=plsc.VectorSubcoreMesh(core_axis_name="c", subcore_axis_name="s"),
    compiler_params=pltpu.CompilerParams(needs_layout_passes=False,
                                         use_tc_tiling_on_sc=False),
    scratch_types=[
        pltpu.VMEM((BPW,), jnp.int32),
        pltpu.VMEM((BPW,), jnp.int32),
        pltpu.VMEM((BPW, K), jnp.int32),
        pltpu.VMEM((BPW * K,), jnp.int32),
        pltpu.VMEM((2, HC, DIM), jnp.float32),
        pltpu.VMEM((BPW * DIM,), jnp.float32),
        pltpu.VMEM((DIM, NUM_REL), jnp.float32),
        pltpu.VMEM((DIM, DIM), jnp.float32),
        pltpu.VMEM((DIM,), jnp.float32),
        pltpu.VMEM((BPW * NUM_REL,), jnp.float32),
        pltpu.VMEM((BPW, K), jnp.int32),
        pltpu.VMEM((BPW, K), jnp.int32),
        pltpu.VMEM((BPW, DIM), jnp.float32),
        pltpu.VMEM((4, K, DIM), jnp.float32),
        pltpu.VMEM((4, K, K), jnp.int32),
        pltpu.VMEM((4, K, K), jnp.int32),
        pltpu.VMEM((4, K * K), jnp.int32),
        pltpu.VMEM((4, K * K, DIM), jnp.float32),
        pltpu.VMEM((K * DIM,), jnp.float32),
        pltpu.VMEM((K, DIM), jnp.float32),
        pltpu.VMEM((BPW * DIM,), jnp.float32),
        pltpu.VMEM((BPW,), jnp.float32),
        pltpu.SemaphoreType.DMA((2,)),
        pltpu.SemaphoreType.DMA((4,)),
        pltpu.SemaphoreType.DMA((4,)),
        pltpu.SemaphoreType.DMA((4,)),
        pltpu.SemaphoreType.DMA((4,)),
        pltpu.SemaphoreType.DMA,
    ],
)(_mkgcn_body)


@jax.jit
def kernel(users, items, entity_table, relation_table, adj_entity,
           adj_relation, user_history, W, b):
    rtt = relation_table.T
    return _mkgcn(users, items, entity_table, rtt, adj_entity, adj_relation,
                  user_history, W, b)
